# baseline (speedup 1.0000x reference)
import jax
import jax.numpy as jnp
from jax.experimental import pallas as pl
from jax.experimental.pallas import tpu as pltpu

_XG = 8
_NBUF = 4


def _feat_body(cl_ref, bb_ref, fx_ref, fy_ref):
    x = cl_ref[0]
    bb = bb_ref[0]
    s = jax.nn.sigmoid(x)
    sb = jnp.minimum(1.0 - s, s)
    ms = jnp.max(s, axis=-1, keepdims=True)
    lp = jnp.log(ms + 1e-8)
    ln = jnp.log(1.0 - ms + 1e-8)
    N, C = x.shape
    z4 = jnp.zeros((N, 4), jnp.float32)
    zc = jnp.zeros((N, C), jnp.float32)
    z1 = jnp.zeros((N, 1), jnp.float32)
    fx_ref[0] = jnp.concatenate(
        [bb, z4, s, zc, sb, zc, lp, z1, ln, z1, ms, z1], axis=1)
    fy_ref[0] = jnp.concatenate(
        [z4, bb, zc, s, zc, sb, z1, lp, z1, ln, z1, ms], axis=1)


def _pair_block(fx_ref, fy_ref, b, x):
    N, W = fx_ref.shape[1], fx_ref.shape[2]
    P = N - 1
    fya = fy_ref[b, :P, :]
    fyb = fy_ref[b, 1:, :]
    rid = jax.lax.broadcasted_iota(jnp.int32, (P, 1), 0)
    fx_row = jnp.broadcast_to(fx_ref[b, pl.ds(x, 1), :], (P, W))
    return jnp.where(rid < x, fya, fyb) + fx_row


def _make_main_body(N, W, KB, S):
    P = N - 1
    ROWS = _XG * P

    def body(fx_ref, fy_ref, out_ref, scratch, sems):
        s = pl.program_id(0)
        k = jax.lax.rem(s, KB)
        b = jax.lax.div(s, KB)
        slot = jax.lax.rem(s, _NBUF)

        def copy_for(step_b, step_k, step_slot):
            return pltpu.make_async_copy(
                scratch.at[step_slot],
                out_ref.at[step_b, pl.ds(step_k * ROWS, ROWS)],
                sems.at[step_slot],
            )

        prev = s - _NBUF

        @pl.when(prev >= 0)
        def _():
            copy_for(jax.lax.div(prev, KB), jax.lax.rem(prev, KB), slot).wait()

        for j in range(_XG):
            scratch[slot, pl.ds(j * P, P), :] = _pair_block(
                fx_ref, fy_ref, b, k * _XG + j)

        copy_for(b, k, slot).start()

        @pl.when(s == S - 1)
        def _():
            for t in range(S - _NBUF + 1, S + 1):
                copy_for(t // KB, t % KB, t % _NBUF).wait()

    return body


def _make_tail_body(N, W, X0):
    P = N - 1

    def body(dummy_ref, fx_ref, fy_ref, out_ref):
        del dummy_ref
        for j in range(_XG):
            x = min(X0 + j, N - 1)
            out_ref[0, pl.ds(j * P, P), :] = _pair_block(
                fx_ref, fy_ref, 0, x)

    return body


def kernel(class_logits, pred_bboxes):
    B, N, C = class_logits.shape
    W = 8 + 4 * C + 6
    P = N * (N - 1)

    fx, fy = pl.pallas_call(
        _feat_body,
        grid=(B,),
        in_specs=[
            pl.BlockSpec((1, N, C), lambda b: (b, 0, 0)),
            pl.BlockSpec((1, N, 4), lambda b: (b, 0, 0)),
        ],
        out_specs=[
            pl.BlockSpec((1, N, W), lambda b: (b, 0, 0)),
            pl.BlockSpec((1, N, W), lambda b: (b, 0, 0)),
        ],
        out_shape=[
            jax.ShapeDtypeStruct((B, N, W), jnp.float32),
            jax.ShapeDtypeStruct((B, N, W), jnp.float32),
        ],
    )(class_logits, pred_bboxes)

    KB = N // _XG
    S = B * KB
    rows = _XG * (N - 1)
    X0 = KB * _XG

    out_main = pl.pallas_call(
        _make_main_body(N, W, KB, S),
        grid=(S,),
        in_specs=[
            pl.BlockSpec((B, N, W), lambda s: (0, 0, 0)),
            pl.BlockSpec((B, N, W), lambda s: (0, 0, 0)),
        ],
        out_specs=pl.BlockSpec(memory_space=pltpu.MemorySpace.HBM),
        out_shape=jax.ShapeDtypeStruct((B, P, W), jnp.float32),
        scratch_shapes=[
            pltpu.VMEM((_NBUF, rows, W), jnp.float32),
            pltpu.SemaphoreType.DMA((_NBUF,)),
        ],
    )(fx, fy)

    out = pl.pallas_call(
        _make_tail_body(N, W, X0),
        grid=(B,),
        in_specs=[
            pl.BlockSpec(memory_space=pltpu.MemorySpace.HBM),
            pl.BlockSpec((1, N, W), lambda b: (b, 0, 0)),
            pl.BlockSpec((1, N, W), lambda b: (b, 0, 0)),
        ],
        out_specs=pl.BlockSpec((1, rows, W), lambda b: (b, KB, 0)),
        out_shape=jax.ShapeDtypeStruct((B, P, W), jnp.float32),
        input_output_aliases={0: 0},
    )(out_main, fx, fy)

    return out


# speedup vs baseline: 1.0021x; 1.0021x over previous
import jax
import jax.numpy as jnp
from jax.experimental import pallas as pl
from jax.experimental.pallas import tpu as pltpu

_XG = 8
_NBUF = 4


def _feat_body(cl_ref, bb_ref, fx_ref, fy_ref):
    x = cl_ref[0]
    bb = bb_ref[0]
    s = jax.nn.sigmoid(x)
    sb = jnp.minimum(1.0 - s, s)
    ms = jnp.max(s, axis=-1, keepdims=True)
    lp = jnp.log(ms + 1e-8)
    ln = jnp.log(1.0 - ms + 1e-8)
    N, C = x.shape
    z4 = jnp.zeros((N, 4), jnp.float32)
    zc = jnp.zeros((N, C), jnp.float32)
    z1 = jnp.zeros((N, 1), jnp.float32)
    fx_ref[0] = jnp.concatenate(
        [bb, z4, s, zc, sb, zc, lp, z1, ln, z1, ms, z1], axis=1)
    fy_ref[0] = jnp.concatenate(
        [z4, bb, zc, s, zc, sb, z1, lp, z1, ln, z1, ms], axis=1)


def _pair_block(fx_ref, fy_ref, b, x):
    N, W = fx_ref.shape[1], fx_ref.shape[2]
    P = N - 1
    fya = fy_ref[b, :P, :]
    fyb = fy_ref[b, 1:, :]
    rid = jax.lax.broadcasted_iota(jnp.int32, (P, 1), 0)
    fx_row = jnp.broadcast_to(fx_ref[b, pl.ds(x, 1), :], (P, W))
    return jnp.where(rid < x, fya, fyb) + fx_row


def _make_main_body(N, W, KB, STEPS):
    P = N - 1
    ROWS = _XG * P

    def body(fx_ref, fy_ref, out_ref, buf0, buf1, buf2, buf3, sems):
        s = pl.program_id(0)
        bufs = (buf0, buf1, buf2, buf3)

        def copy_for(i, chunk_b, chunk_k):
            return pltpu.make_async_copy(
                bufs[i],
                out_ref.at[chunk_b, pl.ds(chunk_k * ROWS, ROWS)],
                sems.at[i],
            )

        for i in range(_NBUF):
            c = s * _NBUF + i
            b = jax.lax.div(c, KB)
            k = jax.lax.rem(c, KB)

            @pl.when(s > 0)
            def _(i=i):
                pc = (s - 1) * _NBUF + i
                copy_for(i, jax.lax.div(pc, KB), jax.lax.rem(pc, KB)).wait()

            for j in range(_XG):
                bufs[i][pl.ds(j * P, P), :] = _pair_block(
                    fx_ref, fy_ref, b, k * _XG + j)

            copy_for(i, b, k).start()

        @pl.when(s == STEPS - 1)
        def _():
            for i in range(_NBUF):
                c = (STEPS - 1) * _NBUF + i
                copy_for(i, c // KB, c % KB).wait()

    return body


def _make_tail_body(N, W, X0):
    P = N - 1

    def body(dummy_ref, fx_ref, fy_ref, out_ref):
        del dummy_ref
        for j in range(_XG):
            x = min(X0 + j, N - 1)
            out_ref[0, pl.ds(j * P, P), :] = _pair_block(
                fx_ref, fy_ref, 0, x)

    return body


def kernel(class_logits, pred_bboxes):
    B, N, C = class_logits.shape
    W = 8 + 4 * C + 6
    P = N * (N - 1)

    fx, fy = pl.pallas_call(
        _feat_body,
        grid=(B,),
        in_specs=[
            pl.BlockSpec((1, N, C), lambda b: (b, 0, 0)),
            pl.BlockSpec((1, N, 4), lambda b: (b, 0, 0)),
        ],
        out_specs=[
            pl.BlockSpec((1, N, W), lambda b: (b, 0, 0)),
            pl.BlockSpec((1, N, W), lambda b: (b, 0, 0)),
        ],
        out_shape=[
            jax.ShapeDtypeStruct((B, N, W), jnp.float32),
            jax.ShapeDtypeStruct((B, N, W), jnp.float32),
        ],
    )(class_logits, pred_bboxes)

    KB = N // _XG
    S = B * KB
    STEPS = S // _NBUF
    rows = _XG * (N - 1)
    X0 = KB * _XG

    out_main = pl.pallas_call(
        _make_main_body(N, W, KB, STEPS),
        grid=(STEPS,),
        in_specs=[
            pl.BlockSpec((B, N, W), lambda s: (0, 0, 0)),
            pl.BlockSpec((B, N, W), lambda s: (0, 0, 0)),
        ],
        out_specs=pl.BlockSpec(memory_space=pltpu.MemorySpace.HBM),
        out_shape=jax.ShapeDtypeStruct((B, P, W), jnp.float32),
        scratch_shapes=[
            pltpu.VMEM((rows, W), jnp.float32),
            pltpu.VMEM((rows, W), jnp.float32),
            pltpu.VMEM((rows, W), jnp.float32),
            pltpu.VMEM((rows, W), jnp.float32),
            pltpu.SemaphoreType.DMA((_NBUF,)),
        ],
    )(fx, fy)

    out = pl.pallas_call(
        _make_tail_body(N, W, X0),
        grid=(B,),
        in_specs=[
            pl.BlockSpec(memory_space=pltpu.MemorySpace.HBM),
            pl.BlockSpec((1, N, W), lambda b: (b, 0, 0)),
            pl.BlockSpec((1, N, W), lambda b: (b, 0, 0)),
        ],
        out_specs=pl.BlockSpec((1, rows, W), lambda b: (b, KB, 0)),
        out_shape=jax.ShapeDtypeStruct((B, P, W), jnp.float32),
        input_output_aliases={0: 0},
    )(out_main, fx, fy)

    return out
